# baseline (speedup 1.0000x reference)
import functools

import jax
import jax.numpy as jnp
from jax.experimental import pallas as pl
from jax.experimental.pallas import tpu as pltpu

BQ = 256
BK = 256
NEG_INF = -1e30


def _flash_kernel(q_ref, k_ref, v_ref, o_ref, *, scale, bq, bk):
    qi = pl.program_id(1)
    q = q_ref[0]
    d = q.shape[-1]

    q_pos = qi * bq + jax.lax.broadcasted_iota(jnp.int32, (bq, bk), 0)

    def body(j, carry):
        acc, m, l = carry
        k = k_ref[0, pl.ds(j * bk, bk), :]
        v = v_ref[0, pl.ds(j * bk, bk), :]
        s = jax.lax.dot_general(
            q, k, (((1,), (1,)), ((), ())),
            preferred_element_type=jnp.float32) * scale
        k_pos = j * bk + jax.lax.broadcasted_iota(jnp.int32, (bq, bk), 1)
        s = jnp.where(k_pos <= q_pos, s, NEG_INF)
        m_new = jnp.maximum(m, jnp.max(s, axis=-1))
        alpha = jnp.exp(m - m_new)
        p = jnp.exp(s - m_new[:, None])
        l_new = l * alpha + jnp.sum(p, axis=-1)
        acc_new = acc * alpha[:, None] + jax.lax.dot_general(
            p, v, (((1,), (0,)), ((), ())),
            preferred_element_type=jnp.float32)
        return acc_new, m_new, l_new

    init = (jnp.zeros((bq, d), jnp.float32),
            jnp.full((bq,), NEG_INF, jnp.float32),
            jnp.zeros((bq,), jnp.float32))
    num_kb = (qi * bq + bq + bk - 1) // bk
    acc, m, l = jax.lax.fori_loop(0, num_kb, body, init)
    o_ref[0] = acc / l[:, None]


@jax.jit
def kernel(query, key, value):
    b, h, s, d = query.shape
    scale = d ** (-0.5)
    q = query.reshape(b * h, s, d)
    k = key.reshape(b * h, s, d)
    v = value.reshape(b * h, s, d)

    grid = (b * h, s // BQ)
    out = pl.pallas_call(
        functools.partial(_flash_kernel, scale=scale, bq=BQ, bk=BK),
        grid=grid,
        in_specs=[
            pl.BlockSpec((1, BQ, d), lambda hh, i: (hh, i, 0)),
            pl.BlockSpec((1, s, d), lambda hh, i: (hh, 0, 0)),
            pl.BlockSpec((1, s, d), lambda hh, i: (hh, 0, 0)),
        ],
        out_specs=pl.BlockSpec((1, BQ, d), lambda hh, i: (hh, i, 0)),
        out_shape=jax.ShapeDtypeStruct((b * h, s, d), jnp.float32),
        compiler_params=pltpu.CompilerParams(
            dimension_semantics=("parallel", "arbitrary"),
        ),
    )(q, k, v)
    return out.reshape(b, h, s, d)
